# baseline (speedup 1.0000x reference)
import jax
import jax.numpy as jnp
from jax.experimental import pallas as pl
from jax.experimental.pallas import tpu as pltpu

H = 512
W = 512


CB = 2


def _stencil_kernel(tab_ref, x_ref, o_ref):
    for ci in range(CB):
        x = x_ref[0, ci]
        a0 = tab_ref[ci, 0, 0]
        a1 = tab_ref[ci, 0, 1]
        a2 = tab_ref[ci, 0, 2]
        a3 = tab_ref[ci, 0, 3]
        a4 = tab_ref[ci, 0, 4]
        a5 = tab_ref[ci, 0, 5]
        b = tab_ref[ci, 0, 6]

        zc1 = jnp.zeros((H, 1), jnp.float32)
        zc2 = jnp.zeros((H, 2), jnp.float32)
        h1 = (jnp.concatenate([x[:, 1:], zc1], 1)
              + jnp.concatenate([zc1, x[:, :-1]], 1))
        h2 = (jnp.concatenate([x[:, 2:], zc2], 1)
              + jnp.concatenate([zc2, x[:, :-2]], 1))

        g0 = a0 * x + a1 * h1 + a3 * h2 + b
        g1 = a1 * x + a2 * h1 + a4 * h2
        g2 = a3 * x + a4 * h1 + a5 * h2

        zr1 = jnp.zeros((1, W), jnp.float32)
        zr2 = jnp.zeros((2, W), jnp.float32)
        out = (g0
               + jnp.concatenate([g1[1:], zr1], 0)
               + jnp.concatenate([zr1, g1[:-1]], 0)
               + jnp.concatenate([g2[2:], zr2], 0)
               + jnp.concatenate([zr2, g2[:-2]], 0))
        o_ref[0, ci] = out


def kernel(x, weight, bias, psi_loc):
    n, c, h, w = x.shape
    full7 = jnp.einsum('kxy,ok->oxy', psi_loc, weight[:, 0, :])
    tab = jnp.stack([
        full7[:, 3, 3],
        full7[:, 3, 4],
        full7[:, 2, 4],
        full7[:, 3, 5],
        full7[:, 2, 5],
        full7[:, 1, 5],
        bias,
        jnp.zeros_like(bias),
    ], axis=-1)
    tab = tab.reshape(c, 1, 8)

    out = pl.pallas_call(
        _stencil_kernel,
        grid=(n, c // CB),
        in_specs=[
            pl.BlockSpec((CB, 1, 8), lambda i, j: (j, 0, 0)),
            pl.BlockSpec((1, CB, h, w), lambda i, j: (i, j, 0, 0)),
        ],
        out_specs=pl.BlockSpec((1, CB, h, w), lambda i, j: (i, j, 0, 0)),
        out_shape=jax.ShapeDtypeStruct((n, c, h, w), jnp.float32),
        compiler_params=pltpu.CompilerParams(
            dimension_semantics=("parallel", "parallel")),
    )(tab, x)
    return out


# speedup vs baseline: 1.0177x; 1.0177x over previous
import jax
import jax.numpy as jnp
from jax.experimental import pallas as pl
from jax.experimental.pallas import tpu as pltpu

H = 512
W = 512


CB = 4


def _stencil_kernel(tab_ref, x_ref, o_ref):
    for ci in range(CB):
        x = x_ref[0, ci]
        a0 = tab_ref[ci, 0, 0]
        a1 = tab_ref[ci, 0, 1]
        a2 = tab_ref[ci, 0, 2]
        a3 = tab_ref[ci, 0, 3]
        a4 = tab_ref[ci, 0, 4]
        a5 = tab_ref[ci, 0, 5]
        b = tab_ref[ci, 0, 6]

        zc1 = jnp.zeros((H, 1), jnp.float32)
        zc2 = jnp.zeros((H, 2), jnp.float32)
        h1 = (jnp.concatenate([x[:, 1:], zc1], 1)
              + jnp.concatenate([zc1, x[:, :-1]], 1))
        h2 = (jnp.concatenate([x[:, 2:], zc2], 1)
              + jnp.concatenate([zc2, x[:, :-2]], 1))

        g0 = a0 * x + a1 * h1 + a3 * h2 + b
        g1 = a1 * x + a2 * h1 + a4 * h2
        g2 = a3 * x + a4 * h1 + a5 * h2

        zr1 = jnp.zeros((1, W), jnp.float32)
        zr2 = jnp.zeros((2, W), jnp.float32)
        out = (g0
               + jnp.concatenate([g1[1:], zr1], 0)
               + jnp.concatenate([zr1, g1[:-1]], 0)
               + jnp.concatenate([g2[2:], zr2], 0)
               + jnp.concatenate([zr2, g2[:-2]], 0))
        o_ref[0, ci] = out


def kernel(x, weight, bias, psi_loc):
    n, c, h, w = x.shape
    full7 = jnp.einsum('kxy,ok->oxy', psi_loc, weight[:, 0, :])
    tab = jnp.stack([
        full7[:, 3, 3],
        full7[:, 3, 4],
        full7[:, 2, 4],
        full7[:, 3, 5],
        full7[:, 2, 5],
        full7[:, 1, 5],
        bias,
        jnp.zeros_like(bias),
    ], axis=-1)
    tab = tab.reshape(c, 1, 8)

    out = pl.pallas_call(
        _stencil_kernel,
        grid=(n, c // CB),
        in_specs=[
            pl.BlockSpec((CB, 1, 8), lambda i, j: (j, 0, 0)),
            pl.BlockSpec((1, CB, h, w), lambda i, j: (i, j, 0, 0)),
        ],
        out_specs=pl.BlockSpec((1, CB, h, w), lambda i, j: (i, j, 0, 0)),
        out_shape=jax.ShapeDtypeStruct((n, c, h, w), jnp.float32),
        compiler_params=pltpu.CompilerParams(
            dimension_semantics=("parallel", "parallel")),
    )(tab, x)
    return out


# speedup vs baseline: 1.0520x; 1.0337x over previous
import jax
import jax.numpy as jnp
from jax.experimental import pallas as pl
from jax.experimental.pallas import tpu as pltpu

H = 512
W = 512


CB = 4


def _stencil_kernel(tab_ref, x_ref, o_ref, s1_ref, s2_ref):
    zb = jnp.zeros((8, W), jnp.float32)
    s1_ref[pl.ds(0, 8), :] = zb
    s1_ref[pl.ds(H + 8, 8), :] = zb
    s2_ref[pl.ds(0, 8), :] = zb
    s2_ref[pl.ds(H + 8, 8), :] = zb
    for ci in range(CB):
        x = x_ref[0, ci]
        a0 = tab_ref[ci, 0, 0]
        a1 = tab_ref[ci, 0, 1]
        a2 = tab_ref[ci, 0, 2]
        a3 = tab_ref[ci, 0, 3]
        a4 = tab_ref[ci, 0, 4]
        a5 = tab_ref[ci, 0, 5]
        b = tab_ref[ci, 0, 6]

        zc1 = jnp.zeros((H, 1), jnp.float32)
        zc2 = jnp.zeros((H, 2), jnp.float32)
        h1 = (jnp.concatenate([x[:, 1:], zc1], 1)
              + jnp.concatenate([zc1, x[:, :-1]], 1))
        h2 = (jnp.concatenate([x[:, 2:], zc2], 1)
              + jnp.concatenate([zc2, x[:, :-2]], 1))

        g0 = a0 * x + a1 * h1 + a3 * h2 + b
        s1_ref[pl.ds(8, H), :] = a1 * x + a2 * h1 + a4 * h2
        s2_ref[pl.ds(8, H), :] = a3 * x + a4 * h1 + a5 * h2

        out = (g0
               + s1_ref[pl.ds(9, H), :]
               + s1_ref[pl.ds(7, H), :]
               + s2_ref[pl.ds(10, H), :]
               + s2_ref[pl.ds(6, H), :])
        o_ref[0, ci] = out


def kernel(x, weight, bias, psi_loc):
    n, c, h, w = x.shape
    full7 = jnp.einsum('kxy,ok->oxy', psi_loc, weight[:, 0, :])
    tab = jnp.stack([
        full7[:, 3, 3],
        full7[:, 3, 4],
        full7[:, 2, 4],
        full7[:, 3, 5],
        full7[:, 2, 5],
        full7[:, 1, 5],
        bias,
        jnp.zeros_like(bias),
    ], axis=-1)
    tab = tab.reshape(c, 1, 8)

    out = pl.pallas_call(
        _stencil_kernel,
        grid=(n, c // CB),
        in_specs=[
            pl.BlockSpec((CB, 1, 8), lambda i, j: (j, 0, 0)),
            pl.BlockSpec((1, CB, h, w), lambda i, j: (i, j, 0, 0)),
        ],
        out_specs=pl.BlockSpec((1, CB, h, w), lambda i, j: (i, j, 0, 0)),
        out_shape=jax.ShapeDtypeStruct((n, c, h, w), jnp.float32),
        compiler_params=pltpu.CompilerParams(
            dimension_semantics=("parallel", "parallel")),
        scratch_shapes=[pltpu.VMEM((h + 16, w), jnp.float32),
                        pltpu.VMEM((h + 16, w), jnp.float32)],
    )(tab, x)
    return out
